# baseline (speedup 1.0000x reference)
import functools

import jax
import jax.numpy as jnp
from jax import lax
from jax.experimental import pallas as pl
from jax.experimental.pallas import tpu as pltpu
from jax.experimental.pallas import tpu_sc as plsc

_L = 16
_NW = 32
_CH = 128


def _router_body(num_tokens, d_model, n_tiles,
                 xflat, posflat, sl_hbm, sigs_hbm, pe_hbm, plate_hbm,
                 sel_out, tgt_out,
                 pos_v, idx_v, x0_v, x1_v, sel_v, tgt_v,
                 sigs_v, pe_v, plate_v, sl_v, tmp_v, sem):
  per = num_tokens // _NW
  nstream = per // _CH
  nchunk = per // _L

  wid = lax.axis_index("s") * 2 + lax.axis_index("c")
  base = wid * per

  pltpu.sync_copy(posflat.at[pl.ds(base, per)], pos_v)
  pltpu.sync_copy(sigs_hbm, sigs_v)
  pltpu.sync_copy(pe_hbm, pe_v)
  pltpu.sync_copy(plate_hbm, plate_v)
  pltpu.sync_copy(sl_hbm, sl_v)

  lane = lax.iota(jnp.int32, _L)
  lane_d = lane * jnp.full((_L,), d_model, jnp.int32)
  one_i = jnp.full((_L,), 1, jnp.int32)
  zero_i = jnp.full((_L,), 0, jnp.int32)
  four_i = jnp.full((_L,), 4, jnp.int32)

  for j in range(nstream):
    for k in range(_CH // _L):
      t0 = (base + j * _CH + k * _L) * d_model
      v = lane_d + jnp.broadcast_to(t0, (_L,))
      idx_v[j, pl.ds(k * _L, _L)] = v
      idx_v[j + nstream, pl.ds(k * _L, _L)] = v + one_i

  copies = []
  for j in range(nstream):
    copies.append(pltpu.async_copy(
        xflat.at[idx_v.at[j]], x0_v.at[pl.ds(j * _CH, _CH)], sem))
    copies.append(pltpu.async_copy(
        xflat.at[idx_v.at[j + nstream]], x1_v.at[pl.ds(j * _CH, _CH)], sem))

  two_i = jnp.full((_L,), 2, jnp.int32)
  half = lax.shift_right_logical(sl_v[...] + one_i, one_i)

  one_f = jnp.full((_L,), 1.0, jnp.float32)
  two_f = jnp.full((_L,), 2.0, jnp.float32)
  zero_f = jnp.full((_L,), 0.0, jnp.float32)
  neg_inf = jnp.full((_L,), -jnp.inf, jnp.float32)

  valid = lane < jnp.full((_L,), n_tiles, jnp.int32)
  row_idx = jnp.where(valid, lane, zero_i)
  acc_e = zero_f
  acc_l = zero_f
  for p in range(_L):
    colp = jnp.full((_L,), p, jnp.int32)
    col = plsc.load_gather(sigs_v, (row_idx, colp))
    th = one_f - two_f / (jnp.exp(col * two_f) + one_f)
    we = plsc.load_gather(pe_v, (colp,))
    wl = plsc.load_gather(plate_v, (colp,))
    acc_e = acc_e + we * th
    acc_l = acc_l + wl * th

  def argmax_splat(svec):
    cur = svec
    for sh in (8, 4, 2, 1):
      tmp_v[...] = cur
      partner = plsc.load_gather(tmp_v, (lane ^ jnp.full((_L,), sh, jnp.int32),))
      cur = jnp.maximum(cur, partner)
    sel = plsc.all_reduce_ffs(svec == cur)
    return jnp.broadcast_to(sel, (_L,))

  e_sel = argmax_splat(jnp.where(valid, acc_e, neg_inf))
  l_sel = argmax_splat(jnp.where(valid, acc_l, neg_inf))

  for h in copies:
    h.wait()

  for c in range(nchunk):
    sl_ix = pl.ds(c * _L, _L)
    p16 = pos_v[sl_ix]
    is_early = p16 < half
    x0 = x0_v[sl_ix]
    x1 = x1_v[sl_ix]
    sel_v[sl_ix] = jnp.where(is_early, e_sel, l_sel)
    tgt_v[sl_ix] = (jnp.where(is_early, zero_i, four_i)
                    + jnp.where(x0 > zero_f, two_i, zero_i)
                    + jnp.where(x1 > zero_f, one_i, zero_i))

  pltpu.sync_copy(sel_v, sel_out.at[pl.ds(base, per)])
  pltpu.sync_copy(tgt_v, tgt_out.at[pl.ds(base, per)])


def kernel(x, positions, seq_len, position_sigs, pos_early, pos_late):
  b, s, d = x.shape
  n = b * s
  t_tiles = position_sigs.shape[0]
  per = n // _NW

  xflat = x.reshape(n * d)
  posflat = positions.reshape(n).astype(jnp.int32)
  sl = jnp.full((_L,), seq_len, dtype=jnp.int32)

  mesh = plsc.VectorSubcoreMesh(core_axis_name="c", subcore_axis_name="s",
                                num_cores=2, num_subcores=16)
  out_i32 = jax.ShapeDtypeStruct((n,), jnp.int32)
  fn = pl.kernel(
      functools.partial(_router_body, n, d, t_tiles),
      out_type=[out_i32, out_i32],
      mesh=mesh,
      compiler_params=pltpu.CompilerParams(needs_layout_passes=False),
      scratch_types=[
          pltpu.VMEM((per,), jnp.int32),
          pltpu.VMEM((2 * (per // _CH), _CH), jnp.int32),
          pltpu.VMEM((per,), jnp.float32),
          pltpu.VMEM((per,), jnp.float32),
          pltpu.VMEM((per,), jnp.int32),
          pltpu.VMEM((per,), jnp.int32),
          pltpu.VMEM((t_tiles, _L), jnp.float32),
          pltpu.VMEM((_L,), jnp.float32),
          pltpu.VMEM((_L,), jnp.float32),
          pltpu.VMEM((_L,), jnp.int32),
          pltpu.VMEM((_L,), jnp.float32),
          pltpu.SemaphoreType.DMA,
      ],
  )
  sel, tgt = fn(xflat, posflat, sl, position_sigs, pos_early, pos_late)
  return sel.reshape(b, s), tgt.reshape(b, s)


# speedup vs baseline: 4.5365x; 4.5365x over previous
import functools

import jax
import jax.numpy as jnp
from jax import lax
from jax.experimental import pallas as pl
from jax.experimental.pallas import tpu as pltpu
from jax.experimental.pallas import tpu_sc as plsc

_L = 16
_NW = 32
_BS = 1024


def _selected_body(num_tokens, n_tiles,
                   posflat, sl_hbm, sigs_hbm, pe_hbm, plate_hbm,
                   sel_out,
                   pos_v, sel_v, sigs_v, pe_v, plate_v, sl_v, tmp_v):
  per = num_tokens // _NW
  nchunk = per // _L

  wid = lax.axis_index("s") * 2 + lax.axis_index("c")
  base = wid * per

  pltpu.sync_copy(posflat.at[pl.ds(base, per)], pos_v)
  pltpu.sync_copy(sigs_hbm, sigs_v)
  pltpu.sync_copy(pe_hbm, pe_v)
  pltpu.sync_copy(plate_hbm, plate_v)
  pltpu.sync_copy(sl_hbm, sl_v)

  lane = lax.iota(jnp.int32, _L)
  one_i = jnp.full((_L,), 1, jnp.int32)
  zero_i = jnp.full((_L,), 0, jnp.int32)
  one_f = jnp.full((_L,), 1.0, jnp.float32)
  two_f = jnp.full((_L,), 2.0, jnp.float32)
  zero_f = jnp.full((_L,), 0.0, jnp.float32)
  neg_inf = jnp.full((_L,), -jnp.inf, jnp.float32)

  half = lax.shift_right_logical(sl_v[...] + one_i, one_i)

  valid = lane < jnp.full((_L,), n_tiles, jnp.int32)
  row_idx = jnp.where(valid, lane, zero_i)
  acc_e = zero_f
  acc_l = zero_f
  for p in range(_L):
    colp = jnp.full((_L,), p, jnp.int32)
    col = plsc.load_gather(sigs_v, (row_idx, colp))
    th = one_f - two_f / (jnp.exp(col * two_f) + one_f)
    we = plsc.load_gather(pe_v, (colp,))
    wl = plsc.load_gather(plate_v, (colp,))
    acc_e = acc_e + we * th
    acc_l = acc_l + wl * th

  def argmax_splat(svec):
    cur = svec
    for sh in (8, 4, 2, 1):
      tmp_v[...] = cur
      partner = plsc.load_gather(tmp_v, (lane ^ jnp.full((_L,), sh, jnp.int32),))
      cur = jnp.maximum(cur, partner)
    sel = plsc.all_reduce_ffs(svec == cur)
    return jnp.broadcast_to(sel, (_L,))

  e_sel = argmax_splat(jnp.where(valid, acc_e, neg_inf))
  l_sel = argmax_splat(jnp.where(valid, acc_l, neg_inf))

  for c in range(nchunk):
    sl_ix = pl.ds(c * _L, _L)
    sel_v[sl_ix] = jnp.where(pos_v[sl_ix] < half, e_sel, l_sel)

  pltpu.sync_copy(sel_v, sel_out.at[pl.ds(base, per)])


def _selected_call(posflat, sl, sigs, pe, plate):
  n = posflat.shape[0]
  t_tiles = sigs.shape[0]
  per = n // _NW
  mesh = plsc.VectorSubcoreMesh(core_axis_name="c", subcore_axis_name="s",
                                num_cores=2, num_subcores=16)
  fn = pl.kernel(
      functools.partial(_selected_body, n, t_tiles),
      out_type=jax.ShapeDtypeStruct((n,), jnp.int32),
      mesh=mesh,
      compiler_params=pltpu.CompilerParams(needs_layout_passes=False),
      scratch_types=[
          pltpu.VMEM((per,), jnp.int32),
          pltpu.VMEM((per,), jnp.int32),
          pltpu.VMEM((t_tiles, _L), jnp.float32),
          pltpu.VMEM((_L,), jnp.float32),
          pltpu.VMEM((_L,), jnp.float32),
          pltpu.VMEM((_L,), jnp.int32),
          pltpu.VMEM((_L,), jnp.float32),
      ],
  )
  return fn(posflat, sl, sigs, pe, plate)


def _targets_body(half_ref, x_ref, pos_ref, tgt_ref):
  xb = x_ref[0]
  lane_ix = lax.broadcasted_iota(jnp.int32, xb.shape, 1)
  w = jnp.where(lane_ix == 0, 2, jnp.where(lane_ix == 1, 1, 0))
  content = jnp.sum(jnp.where(xb > 0.0, w, 0), axis=-1)
  pos = pos_ref[0]
  is_late = pos >= half_ref[0]
  tgt_ref[0] = is_late.astype(jnp.int32) * 4 + content.reshape(8, _BS // 8)


def _targets_call(x3, pos3, half):
  nblk = x3.shape[0]
  out3 = jax.ShapeDtypeStruct(pos3.shape, jnp.int32)
  return pl.pallas_call(
      _targets_body,
      grid=(nblk,),
      in_specs=[
          pl.BlockSpec(memory_space=pltpu.SMEM),
          pl.BlockSpec((1, _BS, 128), lambda i: (i, 0, 0)),
          pl.BlockSpec((1, 8, _BS // 8), lambda i: (i, 0, 0)),
      ],
      out_specs=pl.BlockSpec((1, 8, _BS // 8), lambda i: (i, 0, 0)),
      out_shape=out3,
  )(half, x3, pos3)


def kernel(x, positions, seq_len, position_sigs, pos_early, pos_late):
  b, s, d = x.shape
  n = b * s

  posflat = positions.reshape(n).astype(jnp.int32)
  sl = jnp.full((_L,), seq_len, dtype=jnp.int32)
  selected = _selected_call(posflat, sl, position_sigs, pos_early, pos_late)

  x3 = x.reshape(n // _BS, _BS, d)
  pos3 = posflat.reshape(n // _BS, 8, _BS // 8)
  half = jnp.full((1,), (jnp.asarray(seq_len, jnp.int32) + 1) // 2,
                  dtype=jnp.int32)
  targets = _targets_call(x3, pos3, half)

  return selected.reshape(b, s), targets.reshape(b, s)


# speedup vs baseline: 6.0978x; 1.3442x over previous
import functools

import jax
import jax.numpy as jnp
from jax import lax
from jax.experimental import pallas as pl
from jax.experimental.pallas import tpu as pltpu
from jax.experimental.pallas import tpu_sc as plsc

_L = 16
_NW = 32
_NPH = 4


def _router_body(num_tokens, d_model, n_tiles,
                 x4, posflat, sl_hbm, sigs_hbm, pe_hbm, plate_hbm,
                 sel_out, tgt_out,
                 xga, xgb, pos_v, sel_v, tgt_v,
                 sigs_v, pe_v, plate_v, sl_v,
                 sem_a, sem_b, sem_p):
  per = num_tokens // _NW
  ngrp = per // 8
  gper = ngrp // _NPH
  tpp = gper * 8
  cpp = tpp // _L

  wid = lax.axis_index("s") * 2 + lax.axis_index("c")
  base = wid * per
  gbase = wid * ngrp

  bufs = (xga, xgb)
  sems = (sem_a, sem_b)

  def fire(ph):
    return pltpu.async_copy(
        x4.at[pl.ds(gbase + ph * gper, gper), :, pl.ds(0, 128)],
        bufs[ph % 2], sems[ph % 2])

  h = [fire(0), fire(1)]
  cp = pltpu.async_copy(posflat.at[pl.ds(base, per)], pos_v, sem_p)
  pltpu.sync_copy(sigs_hbm, sigs_v)
  pltpu.sync_copy(pe_hbm, pe_v)
  pltpu.sync_copy(plate_hbm, plate_v)
  pltpu.sync_copy(sl_hbm, sl_v)

  lane = lax.iota(jnp.int32, _L)
  one_i = jnp.full((_L,), 1, jnp.int32)
  zero_i = jnp.full((_L,), 0, jnp.int32)
  two_i = jnp.full((_L,), 2, jnp.int32)
  four_i = jnp.full((_L,), 4, jnp.int32)
  seven_i = jnp.full((_L,), 7, jnp.int32)
  three_i = jnp.full((_L,), 3, jnp.int32)
  one_f = jnp.full((_L,), 1.0, jnp.float32)
  two_f = jnp.full((_L,), 2.0, jnp.float32)
  zero_f = jnp.full((_L,), 0.0, jnp.float32)
  neg_inf = jnp.full((_L,), -jnp.inf, jnp.float32)

  half = lax.shift_right_logical(sl_v[...] + one_i, one_i)

  valid = lane < jnp.full((_L,), n_tiles, jnp.int32)
  row_idx = jnp.where(valid, lane, zero_i)
  acc_e = zero_f
  acc_l = zero_f
  for p in range(_L):
    colp = jnp.full((_L,), p, jnp.int32)
    col = plsc.load_gather(sigs_v, (row_idx, colp))
    th = one_f - two_f / (jnp.exp(col * two_f) + one_f)
    we = plsc.load_gather(pe_v, (colp,))
    wl = plsc.load_gather(plate_v, (colp,))
    acc_e = acc_e + we * th
    acc_l = acc_l + wl * th

  gd = lax.GatherDimensionNumbers(
      offset_dims=(), collapsed_slice_dims=(0,), start_index_map=(0,))

  def argmax_splat(svec):
    cur = svec
    for sh in (8, 4, 2, 1):
      perm = lane ^ jnp.full((_L,), sh, jnp.int32)
      partner = lax.gather(cur, perm[:, None], gd, slice_sizes=(1,),
                           mode=lax.GatherScatterMode.PROMISE_IN_BOUNDS)
      cur = jnp.maximum(cur, partner)
    sel = plsc.all_reduce_ffs(svec == cur)
    return jnp.broadcast_to(sel, (_L,))

  e_sel = argmax_splat(jnp.where(valid, acc_e, neg_inf))
  l_sel = argmax_splat(jnp.where(valid, acc_l, neg_inf))

  cp.wait()

  for ph in range(_NPH):
    h[ph % 2].wait()
    buf = bufs[ph % 2]
    for c in range(cpp):
      sl_ix = pl.ds(ph * tpp + c * _L, _L)
      tl = jnp.full((_L,), c * _L, jnp.int32) + lane
      gi = lax.shift_right_logical(tl, three_i)
      ri = tl & seven_i
      x0 = plsc.load_gather(buf, (gi, ri, zero_i))
      x1 = plsc.load_gather(buf, (gi, ri, one_i))
      is_early = pos_v[sl_ix] < half
      sel_v[sl_ix] = jnp.where(is_early, e_sel, l_sel)
      tgt_v[sl_ix] = (jnp.where(is_early, zero_i, four_i)
                      + jnp.where(x0 > zero_f, two_i, zero_i)
                      + jnp.where(x1 > zero_f, one_i, zero_i))
    if ph + 2 < _NPH:
      h[ph % 2] = fire(ph + 2)

  pltpu.sync_copy(sel_v, sel_out.at[pl.ds(base, per)])
  pltpu.sync_copy(tgt_v, tgt_out.at[pl.ds(base, per)])


def kernel(x, positions, seq_len, position_sigs, pos_early, pos_late):
  b, s, d = x.shape
  n = b * s
  t_tiles = position_sigs.shape[0]
  per = n // _NW
  gper = per // 8 // _NPH

  x4 = x.reshape(n // 8, 8, d)
  posflat = positions.reshape(n).astype(jnp.int32)
  sl = jnp.full((_L,), seq_len, dtype=jnp.int32)

  mesh = plsc.VectorSubcoreMesh(core_axis_name="c", subcore_axis_name="s",
                                num_cores=2, num_subcores=16)
  out_i32 = jax.ShapeDtypeStruct((n,), jnp.int32)
  fn = pl.kernel(
      functools.partial(_router_body, n, d, t_tiles),
      out_type=[out_i32, out_i32],
      mesh=mesh,
      compiler_params=pltpu.CompilerParams(needs_layout_passes=False),
      scratch_types=[
          pltpu.VMEM((gper, 8, 128), jnp.float32),
          pltpu.VMEM((gper, 8, 128), jnp.float32),
          pltpu.VMEM((per,), jnp.int32),
          pltpu.VMEM((per,), jnp.int32),
          pltpu.VMEM((per,), jnp.int32),
          pltpu.VMEM((t_tiles, _L), jnp.float32),
          pltpu.VMEM((_L,), jnp.float32),
          pltpu.VMEM((_L,), jnp.float32),
          pltpu.VMEM((_L,), jnp.int32),
          pltpu.SemaphoreType.DMA,
          pltpu.SemaphoreType.DMA,
          pltpu.SemaphoreType.DMA,
      ],
  )
  sel, tgt = fn(x4, posflat, sl, position_sigs, pos_early, pos_late)
  return sel.reshape(b, s), tgt.reshape(b, s)
